# baseline (speedup 1.0000x reference)
import functools

import jax
import jax.numpy as jnp
from jax import lax
from jax.experimental import pallas as pl
from jax.experimental.pallas import tpu as pltpu
from jax.experimental.pallas import tpu_sc as plsc

NC = 1
NS = 16
NW = NC * NS
L = 16

NM, NU = 20000, 5000
NM_PAD, NU_PAD = 20480, 5120
CM, CU = NM_PAD // NW, NU_PAD // NW
VM, VU = CM // L, CU // L
RUNM, RUNU = 4 * CM, 4 * CU
GRP = 2 * CM + 2 * CU

_F = jnp.float32
_I = jnp.int32


def _build():
  mesh = plsc.VectorSubcoreMesh(
      core_axis_name="c", subcore_axis_name="s",
      num_cores=NC, num_subcores=NS)

  @functools.partial(
      pl.kernel,
      out_type=jax.ShapeDtypeStruct((NW, L), _F),
      mesh=mesh,
      scratch_types=[pltpu.VMEM((RUNM,), _I), pltpu.VMEM((RUNU,), _I),
                     pltpu.VMEM((GRP,), _F), pltpu.VMEM((GRP,), _F),
                     pltpu.VMEM((L,), _F), pltpu.SemaphoreType.DMA],
  )
  def run(pred_hbm, tgt_hbm, mi_hbm, ui_hbm, out_hbm,
          civm, civu, vp, vt, acc_v, sem):
    wid = lax.axis_index("s") * NC + lax.axis_index("c")
    lanes = lax.iota(_I, L)

    groups = ((pred_hbm, vp), (tgt_hbm, vt))
    cps = [pltpu.async_copy(
        mi_hbm.at[pl.ds(wid * RUNM + g * 2 * CM, 2 * CM)],
        civm.at[pl.ds(g * 2 * CM, 2 * CM)], sem) for g in range(2)]
    cpu = pltpu.async_copy(ui_hbm.at[pl.ds(wid * RUNU, RUNU)], civu, sem)

    gps = []
    for g, (tab, vv) in enumerate(groups):
      cps[g].wait()
      for k in range(CM // 320):
        off = k * 640
        gps.append(pltpu.async_copy(
            tab.at[civm.at[pl.ds(g * 2 * CM + off, 640)]],
            vv.at[pl.ds(off, 640)], sem))
    cpu.wait()
    for g, (tab, vv) in enumerate(groups):
      gps.append(pltpu.async_copy(
          tab.at[civu.at[pl.ds(g * 2 * CU, 2 * CU)]],
          vv.at[pl.ds(2 * CM, 2 * CU)], sem))
    for g in gps:
      g.wait()

    def term(va, oa, vb, ob, nvec, ch, n_real):
      base = wid * ch
      def body_fast(j, acc):
        o = j * L
        d = va[pl.ds(oa + o, L)] - vb[pl.ds(ob + o, L)]
        return acc + d * d
      def body_masked(j, acc):
        o = j * L
        d = va[pl.ds(oa + o, L)] - vb[pl.ds(ob + o, L)]
        pos = base + o + lanes
        return acc + jnp.where(pos < n_real, d * d, jnp.zeros_like(d))
      z = jnp.zeros((L,), _F)
      full = base + ch <= n_real

      @pl.when(full)
      def _():
        acc_v[...] = lax.fori_loop(0, nvec, body_fast, z, unroll=4)

      @pl.when(jnp.logical_not(full))
      def _():
        acc_v[...] = lax.fori_loop(0, nvec, body_masked, z, unroll=4)

      return acc_v[...]

    t_b = term(vp, 0, vt, 0, VM, CM, NM)
    t_d = term(vp, CM, vt, CM, VM, CM, NM)
    t_up = term(vp, 2 * CM, vp, 2 * CM + CU, VU, CU, NU)
    t_ut = term(vt, 2 * CM, vt, 2 * CM + CU, VU, CU, NU)
    acc_v[...] = 2.0 * (t_b + t_d) + (t_up + t_ut)
    pltpu.sync_copy(acc_v, out_hbm.at[wid])

  return run


_run = _build()

_LIN_W = jnp.array([[16384.0], [128.0], [1.0]], jnp.float32)


def _lin4(lists, npad):
  c = jnp.stack(lists)
  i = (c.astype(jnp.float32) @ _LIN_W)[..., 0].astype(jnp.int32)
  i = jnp.pad(i, ((0, 0), (0, npad - i.shape[1])))
  ch = npad // NW
  return i.reshape(2, 2, NW, ch).transpose(2, 0, 1, 3).reshape(-1)


def kernel(pred_field, tgt_field,
           matched_pred_birth, matched_pred_death,
           matched_tgt_birth, matched_tgt_death,
           unmatched_pred_birth, unmatched_pred_death,
           unmatched_tgt_birth, unmatched_tgt_death):
  mi = _lin4([matched_pred_birth, matched_pred_death,
              matched_tgt_birth, matched_tgt_death], NM_PAD)
  ui = _lin4([unmatched_pred_birth, unmatched_pred_death,
              unmatched_tgt_birth, unmatched_tgt_death], NU_PAD)
  out = _run(pred_field.reshape(-1), tgt_field.reshape(-1), mi, ui)
  return jnp.sum(out).reshape(1)
